# baseline (speedup 1.0000x reference)
import functools

import jax
import jax.numpy as jnp
from jax import lax
from jax.experimental import pallas as pl
from jax.experimental.pallas import tpu as pltpu
from jax.experimental.pallas import tpu_sc as plsc

EMBED_DIM = 16
BATCH = 16384
LANES = 16

_info = plsc.get_sparse_core_info()
_NC, _NS = _info.num_cores, _info.num_subcores
_NW = _NC * _NS
_BPW = BATCH // _NW
_WAVE = 16
_NWAVE = _BPW // _WAVE
_NITER = _NWAVE // 2

_mesh = plsc.VectorSubcoreMesh(core_axis_name="c", subcore_axis_name="s")


@functools.partial(
    pl.kernel,
    mesh=_mesh,
    out_type=jax.ShapeDtypeStruct((EMBED_DIM, BATCH), jnp.float32),
    scratch_types=[
        pltpu.VMEM((_BPW,), jnp.int32),
        pltpu.VMEM((3, _WAVE, EMBED_DIM, 128), jnp.float32),
        pltpu.VMEM((EMBED_DIM, _BPW), jnp.float32),
        pltpu.SemaphoreType.DMA,
        pltpu.SemaphoreType.DMA,
        pltpu.SemaphoreType.DMA,
    ],
    compiler_params=pltpu.CompilerParams(needs_layout_passes=False),
)
def _gather_kernel(table_hbm, idx_hbm, out_hbm, idx_v, blk_v, out_v, s0, s1, s2):
    wid = lax.axis_index("s") * _NC + lax.axis_index("c")
    base = wid * _BPW
    pltpu.sync_copy(idx_hbm.at[pl.ds(base, _BPW)], idx_v)
    lane = lax.iota(jnp.int32, LANES)

    def wave_idx(w):
        return idx_v[pl.ds(w * _WAVE, _WAVE)]

    def wave_offs(vs):
        return lax.shift_left(lax.shift_right_logical(vs, 7), 7)

    def fire(w, buf, sem):
        offs = wave_offs(wave_idx(w))
        for t in range(_WAVE):
            off = pl.multiple_of(offs[t], 128)
            pltpu.make_async_copy(
                table_hbm.at[:, pl.ds(off, 128)], blk_v.at[buf, t], sem
            ).start()

    def drain_extract(w, buf, sem):
        for t in range(_WAVE):
            pltpu.make_async_copy(
                table_hbm.at[:, pl.ds(0, 128)], blk_v.at[buf, t], sem
            ).wait()
        vs = wave_idx(w)
        cols = vs - wave_offs(vs)
        bvec = jnp.full((LANES,), buf, jnp.int32)
        for d in range(EMBED_DIM):
            vec = plsc.load_gather(
                blk_v, [bvec, lane, jnp.full((LANES,), d, jnp.int32), cols]
            )
            out_v[d, pl.ds(w * _WAVE, _WAVE)] = vec

    fire(0, 0, s0)
    fire(1, 1, s1)

    def body(j, carry):
        w = 3 * j
        fire(w + 2, 2, s2)
        drain_extract(w, 0, s0)
        fire(w + 3, 0, s0)
        drain_extract(w + 1, 1, s1)
        fire(w + 4, 1, s1)
        drain_extract(w + 2, 2, s2)
        return carry

    lax.fori_loop(0, (_NWAVE - 2) // 3, body, 0)
    drain_extract(_NWAVE - 2, 0, s0)
    drain_extract(_NWAVE - 1, 1, s1)
    pltpu.sync_copy(out_v, out_hbm.at[:, pl.ds(base, _BPW)])


def kernel(g, h, r, norm, node_attri):
    table_t = node_attri.T
    idx = h.reshape(BATCH)
    out_t = _gather_kernel(table_t, idx)
    return out_t.T


# speedup vs baseline: 1.0537x; 1.0537x over previous
import functools

import jax
import jax.numpy as jnp
from jax import lax
from jax.experimental import pallas as pl
from jax.experimental.pallas import tpu as pltpu
from jax.experimental.pallas import tpu_sc as plsc

EMBED_DIM = 16
BATCH = 16384
LANES = 16

_info = plsc.get_sparse_core_info()
_NC, _NS = _info.num_cores, _info.num_subcores
_NW = _NC * _NS
_BPW = BATCH // _NW
_WAVE = 16
_NWAVE = _BPW // _WAVE

_mesh = plsc.VectorSubcoreMesh(core_axis_name="c", subcore_axis_name="s")


@functools.partial(
    pl.kernel,
    mesh=_mesh,
    out_type=jax.ShapeDtypeStruct((EMBED_DIM, BATCH), jnp.float32),
    scratch_types=[
        pltpu.VMEM((_BPW,), jnp.int32),
        pltpu.VMEM((3, EMBED_DIM, _WAVE * 128), jnp.float32),
        pltpu.VMEM((EMBED_DIM, _BPW), jnp.float32),
        pltpu.SemaphoreType.DMA,
        pltpu.SemaphoreType.DMA,
        pltpu.SemaphoreType.DMA,
    ],
    compiler_params=pltpu.CompilerParams(needs_layout_passes=False),
)
def _gather_kernel(table_hbm, idx_hbm, out_hbm, idx_v, blk_v, out_v, s0, s1, s2):
    wid = lax.axis_index("s") * _NC + lax.axis_index("c")
    base = wid * _BPW
    pltpu.sync_copy(idx_hbm.at[pl.ds(base, _BPW)], idx_v)
    lane = lax.iota(jnp.int32, LANES)

    def wave_idx(w):
        return idx_v[pl.ds(w * _WAVE, _WAVE)]

    def wave_offs(vs):
        return lax.shift_left(lax.shift_right_logical(vs, 7), 7)

    tcols = lax.iota(jnp.int32, LANES) * 128

    def fire(w, buf, sem):
        offs = wave_offs(wave_idx(w))
        for t in range(_WAVE):
            off = pl.multiple_of(offs[t], 128)
            pltpu.make_async_copy(
                table_hbm.at[:, pl.ds(off, 128)],
                blk_v.at[buf, :, pl.ds(t * 128, 128)],
                sem,
            ).start()

    def drain_extract(w, buf, sem):
        pltpu.make_async_copy(
            table_hbm.at[:, pl.ds(0, _WAVE * 128)], blk_v.at[buf], sem
        ).wait()
        vs = wave_idx(w)
        cols = (vs - wave_offs(vs)) + tcols
        bvec = jnp.full((LANES,), buf, jnp.int32)
        for d in range(EMBED_DIM):
            vec = plsc.load_gather(
                blk_v, [bvec, jnp.full((LANES,), d, jnp.int32), cols]
            )
            out_v[d, pl.ds(w * _WAVE, _WAVE)] = vec

    fire(0, 0, s0)
    fire(1, 1, s1)

    def body(j, carry):
        w = 3 * j
        fire(w + 2, 2, s2)
        drain_extract(w, 0, s0)
        fire(w + 3, 0, s0)
        drain_extract(w + 1, 1, s1)
        fire(w + 4, 1, s1)
        drain_extract(w + 2, 2, s2)
        return carry

    lax.fori_loop(0, (_NWAVE - 2) // 3, body, 0)
    drain_extract(_NWAVE - 2, 0, s0)
    drain_extract(_NWAVE - 1, 1, s1)
    pltpu.sync_copy(out_v, out_hbm.at[:, pl.ds(base, _BPW)])


def kernel(g, h, r, norm, node_attri):
    table_t = node_attri.T
    idx = h.reshape(BATCH)
    out_t = _gather_kernel(table_t, idx)
    return out_t.T
